# baseline (speedup 1.0000x reference)
import functools

import jax
import jax.numpy as jnp
from jax import lax
from jax.experimental import pallas as pl
from jax.experimental.pallas import tpu as pltpu
from jax.experimental.pallas import tpu_sc as plsc

HIDDEN = 768
MAX_POS = 77
BATCH = 4096
SEQ = 77
SEQ_PAD = 80
NW = 32
SPW = BATCH // NW
CHUNK = 16
CPS = SEQ_PAD // CHUNK
TOTAL_ROWS = BATCH * SEQ
LANES = 16
VPR = HIDDEN // LANES


def _embed_kernel(ids_hbm, tab_hbm, pos_hbm, out_hbm,
                  idx_v, pos_v, buf_a, buf_b, sg_a, sg_b, so_a, so_b):
    wid = lax.axis_index("s") * 2 + lax.axis_index("c")
    pltpu.sync_copy(pos_hbm, pos_v)
    pltpu.sync_copy(ids_hbm.at[pl.ds(wid * SPW * SEQ_PAD, SPW * SEQ_PAD)],
                    idx_v)
    lanes = lax.iota(jnp.int32, LANES)

    def start_gather(k, buf, sem):
        s = k // CPS
        ci = k - s * CPS
        ids_vec = idx_v[pl.ds(s * SEQ_PAD + ci * CHUNK, CHUNK)]
        cp = pltpu.async_copy(tab_hbm.at[ids_vec], buf, sem)
        return cp, s, ci

    def add_pos(buf, ci):
        def row(r, carry):
            pbase = (ci * CHUNK + r) * HIDDEN
            for c in range(VPR):
                buf[r, pl.ds(c * LANES, LANES)] = (
                    buf[r, pl.ds(c * LANES, LANES)]
                    + pos_v[pl.ds(pbase + c * LANES, LANES)])
            return carry
        lax.fori_loop(0, CHUNK, row, 0)

    def start_scatter(buf, s, ci, sem):
        base = (wid * SPW + s) * SEQ + ci * CHUNK
        tgt = base + lanes
        tgt = jnp.where(tgt >= TOTAL_ROWS, tgt - TOTAL_ROWS, tgt)
        return pltpu.async_copy(buf, out_hbm.at[tgt], sem)

    def body(t, carry):
        k0 = 2 * t
        cp_a, s0, c0 = start_gather(k0, buf_a, sg_a)
        cp_b, s1, c1 = start_gather(k0 + 1, buf_b, sg_b)
        cp_a.wait()
        add_pos(buf_a, c0)
        sc_a = start_scatter(buf_a, s0, c0, so_a)
        cp_b.wait()
        add_pos(buf_b, c1)
        sc_b = start_scatter(buf_b, s1, c1, so_b)
        sc_a.wait()
        sc_b.wait()
        return carry

    lax.fori_loop(0, SPW * CPS // 2, body, 0)


def kernel(input_ids, token_embedding, position_embedding):
    ids = input_ids.astype(jnp.int32)
    ids_pad = jnp.concatenate([ids, jnp.roll(ids, -1, axis=0)[:, :3]],
                              axis=1).reshape(-1)
    pos_ext = jnp.concatenate(
        [position_embedding, position_embedding[:3]], axis=0).reshape(-1)

    mesh = plsc.VectorSubcoreMesh(core_axis_name="c", subcore_axis_name="s")
    run = functools.partial(
        pl.kernel,
        mesh=mesh,
        out_type=jax.ShapeDtypeStruct((TOTAL_ROWS, HIDDEN), jnp.float32),
        scratch_types=[
            pltpu.VMEM((SPW * SEQ_PAD,), jnp.int32),
            pltpu.VMEM((SEQ_PAD * HIDDEN,), jnp.float32),
            pltpu.VMEM((CHUNK, HIDDEN), jnp.float32),
            pltpu.VMEM((CHUNK, HIDDEN), jnp.float32),
            pltpu.SemaphoreType.DMA,
            pltpu.SemaphoreType.DMA,
            pltpu.SemaphoreType.DMA,
            pltpu.SemaphoreType.DMA,
        ],
    )(_embed_kernel)
    out = run(ids_pad, token_embedding, pos_ext)
    return out.reshape(BATCH, SEQ, HIDDEN)


# speedup vs baseline: 1.3979x; 1.3979x over previous
import functools

import jax
import jax.numpy as jnp
from jax import lax
from jax.experimental import pallas as pl
from jax.experimental.pallas import tpu as pltpu
from jax.experimental.pallas import tpu_sc as plsc

HIDDEN = 768
MAX_POS = 77
BATCH = 4096
SEQ = 77
SEQ_PAD = 80
NW = 32
SPW = BATCH // NW
CHUNK = 16
CPS = SEQ_PAD // CHUNK
NCHUNK = SPW * CPS
TOTAL_ROWS = BATCH * SEQ
LANES = 16
VPR = HIDDEN // LANES
NSLOT = 4


def _embed_kernel(ids_hbm, tab_hbm, pos_hbm, out_hbm,
                  idx_v, pos_v, b0, b1, b2, b3,
                  g0, g1, g2, g3, o0, o1, o2, o3):
    bufs = (b0, b1, b2, b3)
    gsem = (g0, g1, g2, g3)
    osem = (o0, o1, o2, o3)
    wid = lax.axis_index("s") * 2 + lax.axis_index("c")
    pltpu.sync_copy(pos_hbm, pos_v)
    pltpu.sync_copy(ids_hbm.at[pl.ds(wid * SPW * SEQ_PAD, SPW * SEQ_PAD)],
                    idx_v)
    lanes = lax.iota(jnp.int32, LANES)

    def start_gather(k, slot):
        s = k // CPS
        ci = k - s * CPS
        ids_vec = idx_v[pl.ds(s * SEQ_PAD + ci * CHUNK, CHUNK)]
        return pltpu.async_copy(tab_hbm.at[ids_vec], bufs[slot], gsem[slot])

    def add_pos(slot, ci):
        buf = bufs[slot]

        def row(r, carry):
            pbase = (ci * CHUNK + r) * HIDDEN
            for c in range(VPR):
                plsc.addupdate(buf.at[r, pl.ds(c * LANES, LANES)],
                               pos_v[pl.ds(pbase + c * LANES, LANES)])
            return carry
        lax.fori_loop(0, CHUNK, row, 0, unroll=2)

    def start_scatter(k, slot, s, ci):
        base = (wid * SPW + s) * SEQ + ci * CHUNK
        tgt = base + lanes
        tgt = jnp.where(tgt >= TOTAL_ROWS, tgt - TOTAL_ROWS, tgt)
        return pltpu.async_copy(bufs[slot], out_hbm.at[tgt], osem[slot])

    cp0 = start_gather(0, 0)
    cp1 = start_gather(1, 1)

    def round_body(t, carry):
        for slot in range(NSLOT):
            k = NSLOT * t + slot
            s = k // CPS
            ci = k - s * CPS
            pltpu.make_async_copy(
                tab_hbm.at[lanes], bufs[slot], gsem[slot]).wait()
            add_pos(slot, ci)
            start_scatter(k, slot, s, ci)
            nslot = (slot + 2) % NSLOT
            if slot < 2:
                @pl.when(t > 0)
                def _wait():
                    pltpu.make_async_copy(
                        bufs[nslot], out_hbm.at[lanes], osem[nslot]).wait()
            else:
                pltpu.make_async_copy(
                    bufs[nslot], out_hbm.at[lanes], osem[nslot]).wait()

            if slot < 2:
                start_gather(k + 2, nslot)
            else:
                @pl.when(t < NCHUNK // NSLOT - 1)
                def _g():
                    start_gather(k + 2, nslot)
        return carry

    lax.fori_loop(0, NCHUNK // NSLOT, round_body, 0)
    pltpu.make_async_copy(bufs[2], out_hbm.at[lanes], osem[2]).wait()
    pltpu.make_async_copy(bufs[3], out_hbm.at[lanes], osem[3]).wait()


def kernel(input_ids, token_embedding, position_embedding):
    ids = input_ids.astype(jnp.int32)
    ids_pad = jnp.concatenate([ids, jnp.roll(ids, -1, axis=0)[:, :3]],
                              axis=1).reshape(-1)
    pos_ext = jnp.concatenate(
        [position_embedding, position_embedding[:3]], axis=0).reshape(-1)

    mesh = plsc.VectorSubcoreMesh(core_axis_name="c", subcore_axis_name="s")
    run = functools.partial(
        pl.kernel,
        mesh=mesh,
        out_type=jax.ShapeDtypeStruct((TOTAL_ROWS, HIDDEN), jnp.float32),
        scratch_types=[
            pltpu.VMEM((SPW * SEQ_PAD,), jnp.int32),
            pltpu.VMEM((SEQ_PAD * HIDDEN,), jnp.float32),
            pltpu.VMEM((CHUNK, HIDDEN), jnp.float32),
            pltpu.VMEM((CHUNK, HIDDEN), jnp.float32),
            pltpu.VMEM((CHUNK, HIDDEN), jnp.float32),
            pltpu.VMEM((CHUNK, HIDDEN), jnp.float32),
        ] + [pltpu.SemaphoreType.DMA] * 8,
    )(_embed_kernel)
    out = run(ids_pad, token_embedding, pos_ext)
    return out.reshape(BATCH, SEQ, HIDDEN)


# speedup vs baseline: 1.5837x; 1.1330x over previous
import functools

import jax
import jax.numpy as jnp
from jax import lax
from jax.experimental import pallas as pl
from jax.experimental.pallas import tpu as pltpu
from jax.experimental.pallas import tpu_sc as plsc

HIDDEN = 768
BATCH = 4096
SEQ = 77
SEQ_PAD = 80
NW = 32
SPW = BATCH // NW
CHUNK = 16
CPS = SEQ_PAD // CHUNK
NCHUNK = SPW * CPS
TOTAL_ROWS = BATCH * SEQ
LANES = 16
VPR = HIDDEN // LANES
NSLOT = 4


def _embed_kernel(ids_hbm, tab_hbm, pos_hbm, out_hbm,
                  idx_v, pos_v, buf_all, gsem, osem):
    wid = lax.axis_index("s") * 2 + lax.axis_index("c")
    lanes = lax.iota(jnp.int32, LANES)
    pltpu.sync_copy(pos_hbm, pos_v)
    pltpu.sync_copy(ids_hbm.at[pl.ds(wid * SPW * SEQ_PAD, SPW * SEQ_PAD)],
                    idx_v)

    def slot_buf(slot):
        return buf_all.at[pl.ds(pl.multiple_of(slot * CHUNK, CHUNK), CHUNK)]

    def start_gather(k, slot):
        s = k // CPS
        ci = k - s * CPS
        ids_vec = idx_v[pl.ds(s * SEQ_PAD + ci * CHUNK, CHUNK)]
        pltpu.async_copy(tab_hbm.at[ids_vec], slot_buf(slot), gsem.at[slot])

    def step(k):
        slot = lax.rem(k, NSLOT)
        s = k // CPS
        ci = k - s * CPS
        buf = slot_buf(slot)
        pltpu.make_async_copy(tab_hbm.at[lanes], buf, gsem.at[slot]).wait()
        pbase = ci * CHUNK * HIDDEN
        for r in range(CHUNK):
            rbase = pbase + r * HIDDEN
            for g in range(0, VPR, 16):
                vals = [pos_v[pl.ds(rbase + (g + c) * LANES, LANES)]
                        for c in range(16)]
                for c in range(16):
                    plsc.addupdate(buf.at[r, pl.ds((g + c) * LANES, LANES)],
                                   vals[c])

        base = (wid * SPW + s) * SEQ + ci * CHUNK
        tgt = base + lanes
        tgt = jnp.where(tgt >= TOTAL_ROWS, tgt - TOTAL_ROWS, tgt)
        pltpu.async_copy(buf, out_hbm.at[tgt], osem.at[slot])

        nslot = lax.rem(k + 2, NSLOT)
        nbuf = slot_buf(nslot)

        @pl.when(k >= 2)
        def _drain():
            pltpu.make_async_copy(nbuf, out_hbm.at[lanes],
                                  osem.at[nslot]).wait()

        @pl.when(k + 2 < NCHUNK)
        def _gather():
            start_gather(k + 2, nslot)

    start_gather(0, 0)
    start_gather(1, 1)

    def body(k, carry):
        step(k)
        return carry

    lax.fori_loop(0, NCHUNK, body, 0)
    pltpu.make_async_copy(slot_buf(lax.rem(NCHUNK - 2, NSLOT)),
                          out_hbm.at[lanes],
                          osem.at[lax.rem(NCHUNK - 2, NSLOT)]).wait()
    pltpu.make_async_copy(slot_buf(lax.rem(NCHUNK - 1, NSLOT)),
                          out_hbm.at[lanes],
                          osem.at[lax.rem(NCHUNK - 1, NSLOT)]).wait()


def kernel(input_ids, token_embedding, position_embedding):
    ids = input_ids.astype(jnp.int32)
    ids_pad = jnp.concatenate([ids, jnp.roll(ids, -1, axis=0)[:, :3]],
                              axis=1).reshape(-1)
    pos_ext = jnp.concatenate(
        [position_embedding, position_embedding[:3]], axis=0).reshape(-1)

    mesh = plsc.VectorSubcoreMesh(core_axis_name="c", subcore_axis_name="s")
    run = functools.partial(
        pl.kernel,
        mesh=mesh,
        out_type=jax.ShapeDtypeStruct((TOTAL_ROWS, HIDDEN), jnp.float32),
        scratch_types=[
            pltpu.VMEM((SPW * SEQ_PAD,), jnp.int32),
            pltpu.VMEM((SEQ_PAD * HIDDEN,), jnp.float32),
            pltpu.VMEM((NSLOT * CHUNK, HIDDEN), jnp.float32),
            pltpu.SemaphoreType.DMA((NSLOT,)),
            pltpu.SemaphoreType.DMA((NSLOT,)),
        ],
    )(_embed_kernel)
    out = run(ids_pad, token_embedding, pos_ext)
    return out.reshape(BATCH, SEQ, HIDDEN)


# speedup vs baseline: 1.8391x; 1.1613x over previous
import functools

import jax
import jax.numpy as jnp
from jax import lax
from jax.experimental import pallas as pl
from jax.experimental.pallas import tpu as pltpu
from jax.experimental.pallas import tpu_sc as plsc

HIDDEN = 768
BATCH = 4096
SEQ = 77
SEQ_PAD = 80
NW = 32
SPW = BATCH // NW
CHUNK = 16
CPS = SEQ_PAD // CHUNK
NCHUNK = SPW * CPS
TOTAL_ROWS = BATCH * SEQ
LANES = 16
VPR = HIDDEN // LANES
NSLOT = 4


def _embed_kernel(ids_hbm, tab_hbm, pos_hbm, out_hbm,
                  idx_v, pos_v, buf_all, gsem, osem):
    wid = lax.axis_index("s") * 2 + lax.axis_index("c")
    lanes = lax.iota(jnp.int32, LANES)
    pltpu.sync_copy(pos_hbm, pos_v)
    pltpu.sync_copy(ids_hbm.at[pl.ds(wid * SPW * SEQ_PAD, SPW * SEQ_PAD)],
                    idx_v)

    def slot_buf(slot):
        return buf_all.at[pl.ds(pl.multiple_of(slot * CHUNK, CHUNK), CHUNK)]

    def start_gather(k, slot):
        s = k // CPS
        ci = k - s * CPS
        ids_vec = idx_v[pl.ds(s * SEQ_PAD + ci * CHUNK, CHUNK)]
        pltpu.async_copy(tab_hbm.at[ids_vec], slot_buf(slot), gsem.at[slot])

    def step(k):
        slot = lax.rem(k, NSLOT)
        s = k // CPS
        ci = k - s * CPS
        buf = slot_buf(slot)
        pltpu.make_async_copy(tab_hbm.at[lanes], buf, gsem.at[slot]).wait()
        pbase = ci * CHUNK * HIDDEN
        for r in range(CHUNK):
            rbase = pbase + r * HIDDEN
            for g in range(0, VPR, 16):
                vals = [pos_v[pl.ds(rbase + (g + c) * LANES, LANES)]
                        for c in range(16)]
                for c in range(16):
                    plsc.addupdate(buf.at[r, pl.ds((g + c) * LANES, LANES)],
                                   vals[c])

        base = (wid * SPW + s) * SEQ_PAD + ci * CHUNK
        tgt = base + lanes
        pltpu.async_copy(buf, out_hbm.at[tgt], osem.at[slot])

        nslot = lax.rem(k + 2, NSLOT)
        nbuf = slot_buf(nslot)

        @pl.when(k >= 2)
        def _drain():
            pltpu.make_async_copy(nbuf, out_hbm.at[lanes],
                                  osem.at[nslot]).wait()

        @pl.when(k + 2 < NCHUNK)
        def _gather():
            start_gather(k + 2, nslot)

    start_gather(0, 0)
    start_gather(1, 1)

    def body(k, carry):
        step(k)
        return carry

    lax.fori_loop(0, NCHUNK, body, 0)
    pltpu.make_async_copy(slot_buf(lax.rem(NCHUNK - 2, NSLOT)),
                          out_hbm.at[lanes],
                          osem.at[lax.rem(NCHUNK - 2, NSLOT)]).wait()
    pltpu.make_async_copy(slot_buf(lax.rem(NCHUNK - 1, NSLOT)),
                          out_hbm.at[lanes],
                          osem.at[lax.rem(NCHUNK - 1, NSLOT)]).wait()


def kernel(input_ids, token_embedding, position_embedding):
    ids = input_ids.astype(jnp.int32)
    ids_pad = jnp.pad(ids, ((0, 0), (0, SEQ_PAD - SEQ))).reshape(-1)
    pos_ext = jnp.pad(position_embedding,
                      ((0, SEQ_PAD - SEQ), (0, 0))).reshape(-1)

    mesh = plsc.VectorSubcoreMesh(core_axis_name="c", subcore_axis_name="s")
    run = functools.partial(
        pl.kernel,
        mesh=mesh,
        out_type=jax.ShapeDtypeStruct((BATCH * SEQ_PAD, HIDDEN), jnp.float32),
        scratch_types=[
            pltpu.VMEM((SPW * SEQ_PAD,), jnp.int32),
            pltpu.VMEM((SEQ_PAD * HIDDEN,), jnp.float32),
            pltpu.VMEM((NSLOT * CHUNK, HIDDEN), jnp.float32),
            pltpu.SemaphoreType.DMA((NSLOT,)),
            pltpu.SemaphoreType.DMA((NSLOT,)),
        ],
    )(_embed_kernel)
    out = run(ids_pad, token_embedding, pos_ext)
    return out.reshape(BATCH, SEQ_PAD, HIDDEN)[:, :SEQ, :]


# speedup vs baseline: 1.9013x; 1.0338x over previous
import functools

import jax
import jax.numpy as jnp
from jax import lax
from jax.experimental import pallas as pl
from jax.experimental.pallas import tpu as pltpu
from jax.experimental.pallas import tpu_sc as plsc

HIDDEN = 768
BATCH = 4096
SEQ = 77
SEQ_PAD = 80
NW = 32
SPW = BATCH // NW
CHUNK = 16
NSEC = SEQ_PAD // CHUNK
LANES = 16
VPR = HIDDEN // LANES


def _embed_kernel(ids_hbm, tab_hbm, pos_hbm, out_hbm, idx_v, pos_v, *rest):
    bufs = rest[:NSEC]
    gsem = rest[NSEC:2 * NSEC]
    osem = rest[2 * NSEC:3 * NSEC]
    wid = lax.axis_index("s") * 2 + lax.axis_index("c")
    lanes = lax.iota(jnp.int32, LANES)
    pltpu.sync_copy(pos_hbm, pos_v)
    pltpu.sync_copy(ids_hbm.at[pl.ds(wid * SPW * SEQ_PAD, SPW * SEQ_PAD)],
                    idx_v)

    def start_gather(s, i):
        ids_vec = idx_v[pl.ds(s * SEQ_PAD + i * CHUNK, CHUNK)]
        pltpu.async_copy(tab_hbm.at[ids_vec], bufs[i], gsem[i])

    def add_rows(buf, nrows, row0):
        def row(r, carry):
            pb = (row0 + r) * HIDDEN
            for g in range(0, VPR, 16):
                vals = [pos_v[pl.ds(pb + (g + c) * LANES, LANES)]
                        for c in range(16)]
                for c in range(16):
                    plsc.addupdate(buf.at[r, pl.ds((g + c) * LANES, LANES)],
                                   vals[c])
            return carry
        lax.fori_loop(0, nrows, row, 0)

    def drain_out(j):
        pltpu.make_async_copy(bufs[j], out_hbm.at[0, pl.ds(0, CHUNK)],
                              osem[j]).wait()

    def body(s, carry):
        gb = wid * SPW + s
        for i in range(NSEC):
            pltpu.make_async_copy(tab_hbm.at[lanes], bufs[i],
                                  gsem[i]).wait()
            add_rows(bufs[i], CHUNK if i < NSEC - 1 else SEQ - 4 * CHUNK,
                     i * CHUNK)
            pltpu.async_copy(bufs[i],
                             out_hbm.at[gb, pl.ds(i * CHUNK, CHUNK)],
                             osem[i])
            if i <= 2:
                @pl.when(s > 0)
                def _drain():
                    drain_out(i + 2)
                start_gather(s, i + 2)
            else:
                @pl.when(s < SPW - 1)
                def _refill():
                    drain_out(i - 3)
                    start_gather(s + 1, i - 3)
        return carry

    start_gather(0, 0)
    start_gather(0, 1)
    lax.fori_loop(0, SPW, body, 0)
    for i in range(NSEC):
        drain_out(i)


def kernel(input_ids, token_embedding, position_embedding):
    ids = input_ids.astype(jnp.int32)
    ids_pad = jnp.pad(ids, ((0, 0), (0, SEQ_PAD - SEQ))).reshape(-1)
    pos_flat = position_embedding.reshape(-1)

    mesh = plsc.VectorSubcoreMesh(core_axis_name="c", subcore_axis_name="s")
    run = functools.partial(
        pl.kernel,
        mesh=mesh,
        out_type=jax.ShapeDtypeStruct((BATCH, SEQ_PAD, HIDDEN), jnp.float32),
        scratch_types=[
            pltpu.VMEM((SPW * SEQ_PAD,), jnp.int32),
            pltpu.VMEM((SEQ * HIDDEN,), jnp.float32),
        ] + [pltpu.VMEM((CHUNK, HIDDEN), jnp.float32)] * NSEC
          + [pltpu.SemaphoreType.DMA] * (2 * NSEC),
    )(_embed_kernel)
    return run(ids_pad, token_embedding, pos_flat)[:, :SEQ, :]
